# Initial kernel scaffold; baseline (speedup 1.0000x reference)
import jax
import jax.numpy as jnp
from jax.experimental import pallas as pl


def kernel(active_entities, side_entities, params):
    raise NotImplementedError("write your pallas kernel here")


# speedup vs baseline: 3.6994x; 3.6994x over previous
import jax
import jax.numpy as jnp
from jax import lax
from jax.experimental import pallas as pl
from jax.experimental.pallas import tpu as pltpu

_E = 256
_NMERGE = 5
_BM = 512

_F_SPECIES, _F_ITEM, _F_ABILITY, _F_GENDER, _F_ACTIVE, _F_FAINTED, _F_HP, \
    _F_MAXHP, _F_LEVEL, _F_MOVEID0, _F_MOVEID1, _F_MOVEID2, _F_MOVEID3, \
    _F_MOVEPP0, _F_MOVEPP1, _F_MOVEPP2, _F_MOVEPP3, _F_STATUS, \
    _F_ITEM_EFFECT, _F_BEING_CALLED_BACK, _F_TRAPPED, _F_NEWLY_SWITCHED, \
    _F_TOXIC_TURNS, _F_SLEEP_TURNS = range(24)

_SP_PAD = 416
_AB_PAD = 80
_IT_PAD = 136
_MV_PAD = 360


def _encode_block(ents_ref, w0_ref, sp_tbl_ref, sp_w_ref, ab_tbl_ref,
                  ab_w_ref, it_tbl_ref, itw1_ref, effw_ref, mv_tbl_ref,
                  mvw1_ref, ppw_ref, bstk_ref, gate_w_ref, gate_b_ref,
                  enc_w_ref, enc_b_ref, ln_s_ref, ln_b_ref, out_ref):
    ents = ents_ref[...]
    bm = ents.shape[0]

    def col(f):
        return ents[:, f:f + 1]

    def onehot(f, k):
        io = lax.broadcasted_iota(jnp.int32, (bm, k), 1)
        return (col(f) == io).astype(jnp.float32)

    def bits(x, nb):
        io = lax.broadcasted_iota(jnp.int32, (bm, nb), 1)
        return ((lax.shift_right_logical(x, io) & 1)).astype(jnp.float32)

    f32 = jnp.float32

    hpf = col(_F_HP).astype(f32)
    maxhpf = jnp.maximum(col(_F_MAXHP), 1).astype(f32)
    hp_ratio = jnp.clip(hpf / maxhpf, 0.0, 1.0)
    hp_token = (1023.0 * hp_ratio).astype(jnp.int32)
    feat0 = jnp.concatenate(
        [hp_ratio,
         onehot(_F_GENDER, 3),
         onehot(_F_STATUS, 8),
         onehot(_F_BEING_CALLED_BACK, 2),
         onehot(_F_TRAPPED, 2),
         onehot(_F_NEWLY_SWITCHED, 2),
         onehot(_F_TOXIC_TURNS, 8),
         onehot(_F_SLEEP_TURNS, 4),
         onehot(_F_FAINTED, 2),
         bits(hp_token, 10),
         bits(col(_F_LEVEL), 7),
         onehot(_F_ACTIVE, 2),
         jnp.zeros((bm, 5), f32)], axis=1)
    x0 = jnp.dot(feat0, w0_ref[...], preferred_element_type=f32) \
        + bstk_ref[0:1, :]

    sp_rows = jnp.dot(onehot(_F_SPECIES, _SP_PAD), sp_tbl_ref[...],
                      preferred_element_type=f32)
    x1 = jnp.dot(sp_rows, sp_w_ref[...], preferred_element_type=f32) \
        + bstk_ref[1:2, :]

    ab_rows = jnp.dot(onehot(_F_ABILITY, _AB_PAD), ab_tbl_ref[...],
                      preferred_element_type=f32)
    x2 = jnp.dot(ab_rows, ab_w_ref[...], preferred_element_type=f32) \
        + bstk_ref[2:3, :]

    it_rows = jnp.dot(onehot(_F_ITEM, _IT_PAD), it_tbl_ref[...],
                      preferred_element_type=f32)
    x3 = jnp.dot(it_rows, itw1_ref[...], preferred_element_type=f32) \
        + jnp.dot(onehot(_F_ITEM_EFFECT, 16), effw_ref[...],
                  preferred_element_type=f32) \
        + bstk_ref[3:4, :]

    cnt = (onehot(_F_MOVEID0, _MV_PAD) + onehot(_F_MOVEID1, _MV_PAD)
           + onehot(_F_MOVEID2, _MV_PAD) + onehot(_F_MOVEID3, _MV_PAD))
    mv_rows = jnp.dot(cnt, mv_tbl_ref[...], preferred_element_type=f32)
    ppcat = jnp.concatenate(
        [bits(col(_F_MOVEPP0), 8), bits(col(_F_MOVEPP1), 8),
         bits(col(_F_MOVEPP2), 8), bits(col(_F_MOVEPP3), 8)], axis=1)
    x4 = jnp.dot(mv_rows, mvw1_ref[...], preferred_element_type=f32) \
        + jnp.dot(ppcat, ppw_ref[...], preferred_element_type=f32) \
        + bstk_ref[4:5, :]

    ys = []
    for i, x in enumerate((x0, x1, x2, x3, x4)):
        mu = jnp.mean(x, axis=1, keepdims=True)
        d = x - mu
        var = jnp.mean(d * d, axis=1, keepdims=True)
        y = d * lax.rsqrt(var + 1e-6) * ln_s_ref[i:i + 1, :] \
            + ln_b_ref[i:i + 1, :]
        ys.append(jnp.maximum(y, 0.0))

    g = gate_b_ref[...]
    for i in range(_NMERGE):
        g = g + jnp.dot(ys[i], gate_w_ref[i], preferred_element_type=f32)
    gs = [g[:, i * _E:(i + 1) * _E] for i in range(_NMERGE)]
    m = gs[0]
    for i in range(1, _NMERGE):
        m = jnp.maximum(m, gs[i])
    es = [jnp.exp(gi - m) for gi in gs]
    tot = es[0] + es[1] + es[2] + es[3] + es[4]
    out = jnp.zeros((bm, _E), f32)
    for i in range(_NMERGE):
        enc = jnp.dot(ys[i], enc_w_ref[i], preferred_element_type=f32) \
            + enc_b_ref[i:i + 1, :]
        out = out + (es[i] / tot) * enc
    out_ref[...] = out


def kernel(active_entities, side_entities, params):
    p = params
    b = active_entities.shape[0]
    n_active = active_entities.shape[1]
    n_side = side_entities.shape[1]
    ents = jnp.concatenate(
        [active_entities.reshape(-1, 24), side_entities.reshape(-1, 24)],
        axis=0)
    m = ents.shape[0]

    f32 = jnp.float32
    w0 = jnp.concatenate(
        [p['onehot_w'], p['hp_w'], p['level_w'], p['active_w'],
         jnp.zeros((5, _E), f32)], axis=0)
    bstk = jnp.stack(
        [p['onehot_b'] + p['hp_b'] + p['level_b'] + p['active_b'],
         p['species_b'], p['ability_b'], p['item_b'], 4.0 * p['moves_b']],
        axis=0)
    sp_tbl = jnp.pad(p['species_tbl'], ((0, _SP_PAD - 413), (0, 0)))
    ab_tbl = jnp.pad(p['ability_tbl'], ((0, _AB_PAD - 78), (0, 0)))
    it_tbl = jnp.pad(p['item_tbl'], ((0, _IT_PAD - 129), (0, 0)))
    mv_tbl = jnp.pad(p['move_tbl'], ((0, _MV_PAD - 355), (0, 0)))
    itw1 = p['item_w'][:64]
    effw = p['item_w'][64:80]
    mvw1 = p['moves_w'][:128]
    ppw6 = jnp.pad(p['moves_w'][128:134], ((0, 2), (0, 0)))
    ppw = jnp.tile(ppw6, (4, 1))
    gate_b = p['gate_b'].sum(axis=0, keepdims=True)

    grid = (m // _BM,)
    full = lambda shape: pl.BlockSpec(shape, lambda i: tuple(0 for _ in shape))
    out = pl.pallas_call(
        _encode_block,
        grid=grid,
        in_specs=[
            pl.BlockSpec((_BM, 24), lambda i: (i, 0)),
            full((56, _E)),
            full((_SP_PAD, 128)), full((128, _E)),
            full((_AB_PAD, 64)), full((64, _E)),
            full((_IT_PAD, 64)), full((64, _E)), full((16, _E)),
            full((_MV_PAD, 128)), full((128, _E)), full((32, _E)),
            full((5, _E)),
            full((5, _E, 5 * _E)), full((1, 5 * _E)),
            full((5, _E, _E)), full((5, _E)),
            full((5, _E)), full((5, _E)),
        ],
        out_specs=pl.BlockSpec((_BM, _E), lambda i: (i, 0)),
        out_shape=jax.ShapeDtypeStruct((m, _E), f32),
        compiler_params=pltpu.CompilerParams(
            dimension_semantics=("parallel",)),
    )(ents, w0, sp_tbl, p['species_w'], ab_tbl, p['ability_w'],
      it_tbl, itw1, effw, mv_tbl, mvw1, ppw, bstk,
      p['gate_w'], gate_b, p['enc_w'], p['enc_b'],
      p['ln_scale'], p['ln_bias'])

    active_embeddings = out[:b * n_active].reshape(b, n_active, _E)
    side_embeddings = out[b * n_active:].reshape(b, n_side, _E)
    side_species = side_entities[..., _F_SPECIES]
    mask = (side_species != 0) | (side_species != 412)
    return active_embeddings, side_embeddings, mask


# speedup vs baseline: 3.9276x; 1.0617x over previous
import jax
import jax.numpy as jnp
from jax import lax
from jax.experimental import pallas as pl
from jax.experimental.pallas import tpu as pltpu

_E = 256
_NMERGE = 5
_BM = 512

_F_SPECIES, _F_ITEM, _F_ABILITY, _F_GENDER, _F_ACTIVE, _F_FAINTED, _F_HP, \
    _F_MAXHP, _F_LEVEL, _F_MOVEID0, _F_MOVEID1, _F_MOVEID2, _F_MOVEID3, \
    _F_MOVEPP0, _F_MOVEPP1, _F_MOVEPP2, _F_MOVEPP3, _F_STATUS, \
    _F_ITEM_EFFECT, _F_BEING_CALLED_BACK, _F_TRAPPED, _F_NEWLY_SWITCHED, \
    _F_TOXIC_TURNS, _F_SLEEP_TURNS = range(24)

_SP_PAD = 416
_AB_PAD = 80
_IT_PAD = 136
_MV_PAD = 360


def _encode_block(ents_ref, w0_ref, sp_tbl_ref, sp_w_ref, ab_tbl_ref,
                  ab_w_ref, it_tbl_ref, itw1_ref, effw_ref, mv_tbl_ref,
                  mvw1_ref, ppw_ref, bstk_ref, gate_w_ref, gate_b_ref,
                  enc_w_ref, enc_b_ref, ln_s_ref, ln_b_ref, out_ref):
    ents = ents_ref[...]
    bm = ents.shape[0]

    def col(f):
        return ents[:, f:f + 1]

    bf16 = jnp.bfloat16

    def onehot(f, k):
        io = lax.broadcasted_iota(jnp.int32, (bm, k), 1)
        return (col(f) == io).astype(bf16)

    def bits(x, nb):
        io = lax.broadcasted_iota(jnp.int32, (bm, nb), 1)
        return ((lax.shift_right_logical(x, io) & 1)).astype(bf16)

    f32 = jnp.float32

    hpf = col(_F_HP).astype(f32)
    maxhpf = jnp.maximum(col(_F_MAXHP), 1).astype(f32)
    hp_ratio = jnp.clip(hpf / maxhpf, 0.0, 1.0)
    hp_token = (1023.0 * hp_ratio).astype(jnp.int32)
    feat0 = jnp.concatenate(
        [hp_ratio.astype(bf16),
         onehot(_F_GENDER, 3),
         onehot(_F_STATUS, 8),
         onehot(_F_BEING_CALLED_BACK, 2),
         onehot(_F_TRAPPED, 2),
         onehot(_F_NEWLY_SWITCHED, 2),
         onehot(_F_TOXIC_TURNS, 8),
         onehot(_F_SLEEP_TURNS, 4),
         onehot(_F_FAINTED, 2),
         bits(hp_token, 10),
         bits(col(_F_LEVEL), 7),
         onehot(_F_ACTIVE, 2),
         jnp.zeros((bm, 5), bf16)], axis=1)
    x0 = jnp.dot(feat0, w0_ref[...], preferred_element_type=f32) \
        + bstk_ref[0:1, :]

    sp_rows = jnp.dot(onehot(_F_SPECIES, _SP_PAD), sp_tbl_ref[...],
                      preferred_element_type=f32).astype(bf16)
    x1 = jnp.dot(sp_rows, sp_w_ref[...], preferred_element_type=f32) \
        + bstk_ref[1:2, :]

    ab_rows = jnp.dot(onehot(_F_ABILITY, _AB_PAD), ab_tbl_ref[...],
                      preferred_element_type=f32).astype(bf16)
    x2 = jnp.dot(ab_rows, ab_w_ref[...], preferred_element_type=f32) \
        + bstk_ref[2:3, :]

    it_rows = jnp.dot(onehot(_F_ITEM, _IT_PAD), it_tbl_ref[...],
                      preferred_element_type=f32).astype(bf16)
    x3 = jnp.dot(it_rows, itw1_ref[...], preferred_element_type=f32) \
        + jnp.dot(onehot(_F_ITEM_EFFECT, 16), effw_ref[...],
                  preferred_element_type=f32) \
        + bstk_ref[3:4, :]

    cnt = (onehot(_F_MOVEID0, _MV_PAD) + onehot(_F_MOVEID1, _MV_PAD)
           + onehot(_F_MOVEID2, _MV_PAD) + onehot(_F_MOVEID3, _MV_PAD))
    mv_rows = jnp.dot(cnt, mv_tbl_ref[...],
                      preferred_element_type=f32).astype(bf16)
    ppcat = jnp.concatenate(
        [bits(col(_F_MOVEPP0), 8), bits(col(_F_MOVEPP1), 8),
         bits(col(_F_MOVEPP2), 8), bits(col(_F_MOVEPP3), 8)], axis=1)
    x4 = jnp.dot(mv_rows, mvw1_ref[...], preferred_element_type=f32) \
        + jnp.dot(ppcat, ppw_ref[...], preferred_element_type=f32) \
        + bstk_ref[4:5, :]

    ys = []
    for i, x in enumerate((x0, x1, x2, x3, x4)):
        mu = jnp.mean(x, axis=1, keepdims=True)
        d = x - mu
        var = jnp.mean(d * d, axis=1, keepdims=True)
        y = d * lax.rsqrt(var + 1e-6) * ln_s_ref[i:i + 1, :] \
            + ln_b_ref[i:i + 1, :]
        ys.append(jnp.maximum(y, 0.0).astype(bf16))

    g = gate_b_ref[...]
    for i in range(_NMERGE):
        g = g + jnp.dot(ys[i], gate_w_ref[i], preferred_element_type=f32)
    gs = [g[:, i * _E:(i + 1) * _E] for i in range(_NMERGE)]
    m = gs[0]
    for i in range(1, _NMERGE):
        m = jnp.maximum(m, gs[i])
    es = [jnp.exp(gi - m) for gi in gs]
    tot = es[0] + es[1] + es[2] + es[3] + es[4]
    out = jnp.zeros((bm, _E), f32)
    for i in range(_NMERGE):
        enc = jnp.dot(ys[i], enc_w_ref[i], preferred_element_type=f32) \
            + enc_b_ref[i:i + 1, :]
        out = out + (es[i] / tot) * enc
    out_ref[...] = out


def kernel(active_entities, side_entities, params):
    p = params
    b = active_entities.shape[0]
    n_active = active_entities.shape[1]
    n_side = side_entities.shape[1]
    ents = jnp.concatenate(
        [active_entities.reshape(-1, 24), side_entities.reshape(-1, 24)],
        axis=0)
    m = ents.shape[0]

    f32 = jnp.float32
    bf16 = jnp.bfloat16
    w0 = jnp.concatenate(
        [p['onehot_w'], p['hp_w'], p['level_w'], p['active_w'],
         jnp.zeros((5, _E), f32)], axis=0).astype(bf16)
    bstk = jnp.stack(
        [p['onehot_b'] + p['hp_b'] + p['level_b'] + p['active_b'],
         p['species_b'], p['ability_b'], p['item_b'], 4.0 * p['moves_b']],
        axis=0)
    sp_tbl = jnp.pad(p['species_tbl'], ((0, _SP_PAD - 413), (0, 0))).astype(bf16)
    ab_tbl = jnp.pad(p['ability_tbl'], ((0, _AB_PAD - 78), (0, 0))).astype(bf16)
    it_tbl = jnp.pad(p['item_tbl'], ((0, _IT_PAD - 129), (0, 0))).astype(bf16)
    mv_tbl = jnp.pad(p['move_tbl'], ((0, _MV_PAD - 355), (0, 0))).astype(bf16)
    sp_w = p['species_w'].astype(bf16)
    ab_w = p['ability_w'].astype(bf16)
    itw1 = p['item_w'][:64].astype(bf16)
    effw = p['item_w'][64:80].astype(bf16)
    mvw1 = p['moves_w'][:128].astype(bf16)
    ppw6 = jnp.pad(p['moves_w'][128:134], ((0, 2), (0, 0)))
    ppw = jnp.tile(ppw6, (4, 1)).astype(bf16)
    gate_w = p['gate_w'].astype(bf16)
    enc_w = p['enc_w'].astype(bf16)
    gate_b = p['gate_b'].sum(axis=0, keepdims=True)

    grid = (m // _BM,)
    full = lambda shape: pl.BlockSpec(shape, lambda i: tuple(0 for _ in shape))
    out = pl.pallas_call(
        _encode_block,
        grid=grid,
        in_specs=[
            pl.BlockSpec((_BM, 24), lambda i: (i, 0)),
            full((56, _E)),
            full((_SP_PAD, 128)), full((128, _E)),
            full((_AB_PAD, 64)), full((64, _E)),
            full((_IT_PAD, 64)), full((64, _E)), full((16, _E)),
            full((_MV_PAD, 128)), full((128, _E)), full((32, _E)),
            full((5, _E)),
            full((5, _E, 5 * _E)), full((1, 5 * _E)),
            full((5, _E, _E)), full((5, _E)),
            full((5, _E)), full((5, _E)),
        ],
        out_specs=pl.BlockSpec((_BM, _E), lambda i: (i, 0)),
        out_shape=jax.ShapeDtypeStruct((m, _E), f32),
        compiler_params=pltpu.CompilerParams(
            dimension_semantics=("parallel",)),
    )(ents, w0, sp_tbl, sp_w, ab_tbl, ab_w,
      it_tbl, itw1, effw, mv_tbl, mvw1, ppw, bstk,
      gate_w, gate_b, enc_w, p['enc_b'],
      p['ln_scale'], p['ln_bias'])

    active_embeddings = out[:b * n_active].reshape(b, n_active, _E)
    side_embeddings = out[b * n_active:].reshape(b, n_side, _E)
    side_species = side_entities[..., _F_SPECIES]
    mask = (side_species != 0) | (side_species != 412)
    return active_embeddings, side_embeddings, mask
